# Initial kernel scaffold; baseline (speedup 1.0000x reference)
import jax
import jax.numpy as jnp
from jax.experimental import pallas as pl


def kernel(inputs, embedding_matrix):
    raise NotImplementedError("write your pallas kernel here")


# speedup vs baseline: 1.5763x; 1.5763x over previous
import functools

import jax
import jax.numpy as jnp
from jax import lax
from jax.experimental import pallas as pl
from jax.experimental.pallas import tpu as pltpu
from jax.experimental.pallas import tpu_sc as plsc

D = 32
NC = 2
NS = 16
NW = NC * NS
CH = 1024


@functools.lru_cache(maxsize=None)
def _make_gather(B: int, V: int):
    assert B % (NW * CH) == 0
    b_per_w = B // NW
    n_ch = b_per_w // CH
    mesh = plsc.VectorSubcoreMesh(core_axis_name="c", subcore_axis_name="s")

    @functools.partial(
        pl.kernel,
        mesh=mesh,
        out_type=jax.ShapeDtypeStruct((B, D), jnp.float32),
        compiler_params=pltpu.CompilerParams(use_tc_tiling_on_sc=False),
        scratch_types=[
            pltpu.VMEM((b_per_w,), jnp.int32),
            pltpu.VMEM((CH, D), jnp.float32),
            pltpu.VMEM((CH, D), jnp.float32),
            pltpu.SemaphoreType.DMA,
            pltpu.SemaphoreType.DMA,
        ],
    )
    def gather_kernel(idx_hbm, table_hbm, out_hbm, idx_v, buf0, buf1, s0, s1):
        wid = lax.axis_index("s") * NC + lax.axis_index("c")
        base = wid * b_per_w
        pltpu.sync_copy(idx_hbm.at[pl.ds(base, b_per_w)], idx_v)
        bufs = (buf0, buf1)
        sems = (s0, s1)
        pending = pltpu.async_copy(table_hbm.at[idx_v.at[pl.ds(0, CH)]], buf0, s0)
        for j in range(n_ch):
            nxt = None
            if j + 1 < n_ch:
                nxt = pltpu.async_copy(
                    table_hbm.at[idx_v.at[pl.ds((j + 1) * CH, CH)]],
                    bufs[(j + 1) % 2],
                    sems[(j + 1) % 2],
                )
            pending.wait()
            pltpu.sync_copy(bufs[j % 2], out_hbm.at[pl.ds(base + j * CH, CH)])
            pending = nxt

    return gather_kernel


def kernel(inputs, embedding_matrix):
    B, F = inputs.shape
    V, d = embedding_matrix.shape
    n = B * F
    b_per_w = n // NW
    idx = inputs.reshape(n).astype(jnp.int32)
    out = _make_gather(n, V)(idx, embedding_matrix)
    return out.reshape(B, F, d)


# speedup vs baseline: 1.5765x; 1.0001x over previous
import functools

import jax
import jax.numpy as jnp
from jax import lax
from jax.experimental import pallas as pl
from jax.experimental.pallas import tpu as pltpu
from jax.experimental.pallas import tpu_sc as plsc

D = 32
NC = 2
NS = 16
NW = NC * NS
CH = 1024
NB = 3


@functools.lru_cache(maxsize=None)
def _make_gather(B: int, V: int):
    assert B % (NW * CH) == 0
    b_per_w = B // NW
    n_ch = b_per_w // CH
    mesh = plsc.VectorSubcoreMesh(core_axis_name="c", subcore_axis_name="s")

    @functools.partial(
        pl.kernel,
        mesh=mesh,
        out_type=jax.ShapeDtypeStruct((B, D), jnp.float32),
        compiler_params=pltpu.CompilerParams(use_tc_tiling_on_sc=False),
        scratch_types=[
            pltpu.VMEM((b_per_w,), jnp.int32),
            [pltpu.VMEM((CH, D), jnp.float32) for _ in range(NB)],
            [pltpu.SemaphoreType.DMA for _ in range(NB)],
            [pltpu.SemaphoreType.DMA for _ in range(NB)],
        ],
    )
    def gather_kernel(idx_hbm, table_hbm, out_hbm, idx_v, bufs, gsems, ssems):
        wid = lax.axis_index("s") * NC + lax.axis_index("c")
        base = wid * b_per_w
        pltpu.sync_copy(idx_hbm.at[pl.ds(base, b_per_w)], idx_v)

        def gather(j, b):
            return pltpu.async_copy(
                table_hbm.at[idx_v.at[pl.ds(j * CH, CH)]], bufs[b], gsems[b]
            )

        g = [None] * NB
        st = [None] * NB
        for b in range(min(NB - 1, n_ch)):
            g[b] = gather(b, b)
        for j in range(n_ch):
            b = j % NB
            jj = j + NB - 1
            if jj < n_ch:
                bb = jj % NB
                if jj >= NB:
                    st[bb].wait()
                g[bb] = gather(jj, bb)
            g[b].wait()
            st[b] = pltpu.async_copy(
                bufs[b], out_hbm.at[pl.ds(base + j * CH, CH)], ssems[b]
            )
        for j in range(max(0, n_ch - NB), n_ch):
            st[j % NB].wait()

    return gather_kernel


def kernel(inputs, embedding_matrix):
    B, F = inputs.shape
    V, d = embedding_matrix.shape
    n = B * F
    b_per_w = n // NW
    idx = inputs.reshape(n).astype(jnp.int32)
    out = _make_gather(n, V)(idx, embedding_matrix)
    return out.reshape(B, F, d)


# speedup vs baseline: 1.5769x; 1.0003x over previous
import functools

import jax
import jax.numpy as jnp
from jax import lax
from jax.experimental import pallas as pl
from jax.experimental.pallas import tpu as pltpu
from jax.experimental.pallas import tpu_sc as plsc

D = 32
NC = 2
NS = 16
NW = NC * NS
CH = 512
NB = 6


@functools.lru_cache(maxsize=None)
def _make_gather(B: int, V: int):
    assert B % (NW * CH) == 0
    b_per_w = B // NW
    n_ch = b_per_w // CH
    mesh = plsc.VectorSubcoreMesh(core_axis_name="c", subcore_axis_name="s")

    @functools.partial(
        pl.kernel,
        mesh=mesh,
        out_type=jax.ShapeDtypeStruct((B, D), jnp.float32),
        compiler_params=pltpu.CompilerParams(use_tc_tiling_on_sc=False),
        scratch_types=[
            pltpu.VMEM((b_per_w,), jnp.int32),
            [pltpu.VMEM((CH, D), jnp.float32) for _ in range(NB)],
            [pltpu.SemaphoreType.DMA for _ in range(NB)],
            [pltpu.SemaphoreType.DMA for _ in range(NB)],
        ],
    )
    def gather_kernel(idx_hbm, table_hbm, out_hbm, idx_v, bufs, gsems, ssems):
        wid = lax.axis_index("s") * NC + lax.axis_index("c")
        base = wid * b_per_w
        pltpu.sync_copy(idx_hbm.at[pl.ds(base, b_per_w)], idx_v)

        def gather(j, b):
            return pltpu.async_copy(
                table_hbm.at[idx_v.at[pl.ds(j * CH, CH)]], bufs[b], gsems[b]
            )

        g = [None] * NB
        st = [None] * NB
        for b in range(min(NB - 1, n_ch)):
            g[b] = gather(b, b)
        for j in range(n_ch):
            b = j % NB
            jj = j + NB - 1
            if jj < n_ch:
                bb = jj % NB
                if jj >= NB:
                    st[bb].wait()
                g[bb] = gather(jj, bb)
            g[b].wait()
            st[b] = pltpu.async_copy(
                bufs[b], out_hbm.at[pl.ds(base + j * CH, CH)], ssems[b]
            )
        for j in range(max(0, n_ch - NB), n_ch):
            st[j % NB].wait()

    return gather_kernel


def kernel(inputs, embedding_matrix):
    B, F = inputs.shape
    V, d = embedding_matrix.shape
    n = B * F
    b_per_w = n // NW
    idx = inputs.reshape(n).astype(jnp.int32)
    out = _make_gather(n, V)(idx, embedding_matrix)
    return out.reshape(B, F, d)
